# baseline (speedup 1.0000x reference)
import functools

import jax
import jax.numpy as jnp
from jax import lax
from jax.experimental import pallas as pl
from jax.experimental.pallas import tpu as pltpu
from jax.experimental.pallas import tpu_sc as plsc

_NUM_CORES = 2
_NUM_SUBCORES = 16
_NUM_WORKERS = _NUM_CORES * _NUM_SUBCORES
_ROWS_PER_TILE = 128
_CHUNKS = ((0, 24), (24, 40), (64, 32), (96, 32))
_MAX_CHUNK_ROWS = 40
_NBUF = 3


@functools.partial(jax.jit, static_argnums=(1, 2))
def _position_embedding(table, seq_len, dim):
    assert seq_len == _NUM_WORKERS * _ROWS_PER_TILE and dim % 128 == 0
    mesh = plsc.VectorSubcoreMesh(
        core_axis_name="c", subcore_axis_name="s", num_cores=_NUM_CORES
    )

    @functools.partial(
        pl.kernel,
        out_type=jax.ShapeDtypeStruct((seq_len, dim), table.dtype),
        mesh=mesh,
        scratch_types=[
            pltpu.VMEM((_NBUF, _MAX_CHUNK_ROWS, dim), table.dtype),
            pltpu.SemaphoreType.DMA((_NBUF,)),
            pltpu.SemaphoreType.DMA((_NBUF,)),
        ],
    )
    def copy_kernel(table_hbm, out_hbm, buf, in_sems, out_sems):
        wid = lax.axis_index("s") * _NUM_CORES + lax.axis_index("c")
        base = wid * _ROWS_PER_TILE

        def fire_in(c):
            off, rows = _CHUNKS[c]
            return pltpu.async_copy(
                table_hbm.at[pl.ds(base + off, rows)],
                buf.at[c % _NBUF, pl.ds(0, rows)],
                in_sems.at[c % _NBUF],
            )

        def fire_out(c):
            off, rows = _CHUNKS[c]
            return pltpu.async_copy(
                buf.at[c % _NBUF, pl.ds(0, rows)],
                out_hbm.at[pl.ds(base + off, rows)],
                out_sems.at[c % _NBUF],
            )

        in_dma = [fire_in(c) for c in range(_NBUF)]
        out_dma = []
        for c in range(_NBUF):
            in_dma[c].wait()
            out_dma.append(fire_out(c))
        out_dma[0].wait()
        fire_in(3).wait()
        out_dma.append(fire_out(3))
        out_dma[1].wait()
        out_dma[2].wait()
        out_dma[3].wait()

    return copy_kernel(table)


def kernel(inputs, table):
    seq_len = inputs.shape[-1]
    return _position_embedding(table, seq_len, table.shape[1])


# speedup vs baseline: 1.0291x; 1.0291x over previous
import functools

import jax
import jax.numpy as jnp
from jax import lax
from jax.experimental import pallas as pl
from jax.experimental.pallas import tpu as pltpu
from jax.experimental.pallas import tpu_sc as plsc

_NUM_CORES = 2
_NUM_SUBCORES = 16
_NUM_WORKERS = _NUM_CORES * _NUM_SUBCORES
_ROWS_PER_TILE = 128
_CHUNKS = ((0, 24), (24, 40), (64, 32), (96, 32))
_MAX_CHUNK_ROWS = 40
_NBUF = 3


@functools.partial(jax.jit, static_argnums=(1, 2))
def _position_embedding(table, seq_len, dim):
    assert seq_len == _NUM_WORKERS * _ROWS_PER_TILE and dim % 128 == 0
    mesh = plsc.VectorSubcoreMesh(
        core_axis_name="c", subcore_axis_name="s", num_cores=_NUM_CORES
    )

    @functools.partial(
        pl.kernel,
        out_type=jax.ShapeDtypeStruct((seq_len, dim), table.dtype),
        mesh=mesh,
        scratch_types=[
            pltpu.VMEM((_NBUF, _MAX_CHUNK_ROWS, dim), table.dtype),
            pltpu.SemaphoreType.DMA((_NBUF,)),
            pltpu.SemaphoreType.DMA((_NBUF,)),
        ],
    )
    def copy_kernel(table_hbm, out_hbm, buf, in_sems, out_sems):
        wid = lax.axis_index("s") * _NUM_CORES + lax.axis_index("c")
        base = wid * _ROWS_PER_TILE

        def fire_in(c):
            off, rows = _CHUNKS[c]
            return pltpu.async_copy(
                table_hbm.at[pl.ds(base + off, rows)],
                buf.at[c % _NBUF, pl.ds(0, rows)],
                in_sems.at[c % _NBUF],
            )

        def fire_out(c):
            off, rows = _CHUNKS[c]
            return pltpu.async_copy(
                buf.at[c % _NBUF, pl.ds(0, rows)],
                out_hbm.at[pl.ds(base + off, rows)],
                out_sems.at[c % _NBUF],
            )

        in_dma = [fire_in(c) for c in range(_NBUF)]
        in_dma[0].wait()
        out0 = fire_out(0)
        out0.wait()
        in3 = fire_in(3)
        in_dma[1].wait()
        out1 = fire_out(1)
        in_dma[2].wait()
        out2 = fire_out(2)
        in3.wait()
        out3 = fire_out(3)
        out1.wait()
        out2.wait()
        out3.wait()

    return copy_kernel(table)


def kernel(inputs, table):
    seq_len = inputs.shape[-1]
    return _position_embedding(table, seq_len, table.shape[1])


# speedup vs baseline: 1.0353x; 1.0060x over previous
import functools

import jax
import jax.numpy as jnp
from jax import lax
from jax.experimental import pallas as pl
from jax.experimental.pallas import tpu as pltpu
from jax.experimental.pallas import tpu_sc as plsc

_NUM_CORES = 2
_NUM_SUBCORES = 16
_NUM_WORKERS = _NUM_CORES * _NUM_SUBCORES
_ROWS_PER_TILE = 128
_CHUNKS = ((0, 24), (24, 40), (64, 32))
_LAST_OFF = 96
_LAST_ROWS = (24, 40)
_ROWS_C0 = _LAST_OFF + _LAST_ROWS[0]
_ROWS_PAIR = 2 * _LAST_OFF + sum(_LAST_ROWS)
_MAX_CHUNK_ROWS = 40
_NBUF = 3


@functools.partial(jax.jit, static_argnums=(1, 2))
def _position_embedding(table, seq_len, dim):
    assert seq_len == _NUM_SUBCORES * _ROWS_PAIR and dim % 128 == 0
    mesh = plsc.VectorSubcoreMesh(
        core_axis_name="c", subcore_axis_name="s", num_cores=_NUM_CORES
    )

    @functools.partial(
        pl.kernel,
        out_type=jax.ShapeDtypeStruct((seq_len, dim), table.dtype),
        mesh=mesh,
        scratch_types=[
            pltpu.VMEM((_NBUF, _MAX_CHUNK_ROWS, dim), table.dtype),
            pltpu.SemaphoreType.DMA((_NBUF,)),
            pltpu.SemaphoreType.DMA((_NBUF,)),
        ],
    )
    def copy_kernel(table_hbm, out_hbm, buf, in_sems, out_sems):
        cid = lax.axis_index("c")
        sid = lax.axis_index("s")
        base = sid * _ROWS_PAIR + cid * _ROWS_C0

        def dma_in(off, rows, b):
            return pltpu.make_async_copy(
                table_hbm.at[pl.ds(base + off, rows)],
                buf.at[b, pl.ds(0, rows)],
                in_sems.at[b],
            )

        def dma_out(off, rows, b):
            return pltpu.make_async_copy(
                buf.at[b, pl.ds(0, rows)],
                out_hbm.at[pl.ds(base + off, rows)],
                out_sems.at[b],
            )

        for b, (off, rows) in enumerate(_CHUNKS):
            dma_in(off, rows, b).start()
        dma_in(*_CHUNKS[0], 0).wait()
        dma_out(*_CHUNKS[0], 0).start()
        dma_out(*_CHUNKS[0], 0).wait()
        for core, rows in enumerate(_LAST_ROWS):
            @pl.when(cid == core)
            def _(rows=rows):
                dma_in(_LAST_OFF, rows, 0).start()
        dma_in(*_CHUNKS[1], 1).wait()
        dma_out(*_CHUNKS[1], 1).start()
        dma_in(*_CHUNKS[2], 2).wait()
        dma_out(*_CHUNKS[2], 2).start()
        for core, rows in enumerate(_LAST_ROWS):
            @pl.when(cid == core)
            def _(rows=rows):
                dma_in(_LAST_OFF, rows, 0).wait()
                dma_out(_LAST_OFF, rows, 0).start()
                dma_out(_LAST_OFF, rows, 0).wait()
        dma_out(*_CHUNKS[1], 1).wait()
        dma_out(*_CHUNKS[2], 2).wait()

    return copy_kernel(table)


def kernel(inputs, table):
    seq_len = inputs.shape[-1]
    return _position_embedding(table, seq_len, table.shape[1])


# speedup vs baseline: 1.0387x; 1.0033x over previous
import functools

import jax
from jax import lax
from jax.experimental import pallas as pl
from jax.experimental.pallas import tpu as pltpu
from jax.experimental.pallas import tpu_sc as plsc

_NUM_CORES = 2
_NUM_SUBCORES = 16
_CHUNKS = ((0, 24), (24, 40), (64, 32))
_LAST_OFF = 96
_LAST_ROWS = (24, 40)
_ROWS_C0 = _LAST_OFF + _LAST_ROWS[0]
_ROWS_PAIR = 2 * _LAST_OFF + sum(_LAST_ROWS)
_MAX_CHUNK_ROWS = 40
_NBUF = 3


@functools.partial(jax.jit, static_argnums=(1, 2))
def _position_embedding(table, seq_len, dim):
    assert seq_len == _NUM_SUBCORES * _ROWS_PAIR and dim % 128 == 0
    mesh = plsc.VectorSubcoreMesh(
        core_axis_name="c", subcore_axis_name="s", num_cores=_NUM_CORES
    )

    @functools.partial(
        pl.kernel,
        out_type=jax.ShapeDtypeStruct((seq_len, dim), table.dtype),
        mesh=mesh,
        scratch_types=[
            pltpu.VMEM((_NBUF, _MAX_CHUNK_ROWS, dim), table.dtype),
            pltpu.SemaphoreType.DMA((_NBUF,)),
            pltpu.SemaphoreType.DMA((_NBUF,)),
        ],
    )
    def copy_kernel(table_hbm, out_hbm, buf, in_sems, out_sems):
        cid = lax.axis_index("c")
        sid = lax.axis_index("s")
        base = sid * _ROWS_PAIR + cid * _ROWS_C0

        def dma_in(off, rows, b):
            return pltpu.make_async_copy(
                table_hbm.at[pl.ds(base + off, rows)],
                buf.at[b, pl.ds(0, rows)],
                in_sems.at[b],
            )

        def dma_out(off, rows, b):
            return pltpu.make_async_copy(
                buf.at[b, pl.ds(0, rows)],
                out_hbm.at[pl.ds(base + off, rows)],
                out_sems.at[b],
            )

        for b, (off, rows) in enumerate(_CHUNKS):
            dma_in(off, rows, b).start()
        dma_in(*_CHUNKS[0], 0).wait()
        dma_out(*_CHUNKS[0], 0).start()
        dma_out(*_CHUNKS[0], 0).wait()
        for core, rows in enumerate(_LAST_ROWS):
            @pl.when(cid == core)
            def _(rows=rows):
                dma_in(_LAST_OFF, rows, 0).start()
        dma_in(*_CHUNKS[1], 1).wait()
        dma_out(*_CHUNKS[1], 1).start()
        dma_in(*_CHUNKS[2], 2).wait()
        dma_out(*_CHUNKS[2], 2).start()
        for core, rows in enumerate(_LAST_ROWS):
            @pl.when(cid == core)
            def _(rows=rows):
                dma_in(_LAST_OFF, rows, 0).wait()
                dma_out(_LAST_OFF, rows, 0).start()
                dma_out(_LAST_OFF, rows, 0).wait()
        dma_out(*_CHUNKS[1], 1).wait()
        dma_out(*_CHUNKS[2], 2).wait()

    return copy_kernel(table)


def kernel(inputs, table):
    seq_len = inputs.shape[-1]
    return _position_embedding(table, seq_len, table.shape[1])
